# baseline (speedup 1.0000x reference)
import functools

import jax
import jax.numpy as jnp
from jax import lax
from jax.experimental import pallas as pl
from jax.experimental.pallas import tpu as pltpu

B, N, K, D = 16, 1024, 64, 128
THR = 0.97
MASK_VAL = -10000.0
PROP = 0.5
SMOOTH = 0.2
T1, T2 = 0.8, 1.2


def _stage1_body(cur_ref, row_ref, idx_ref, mask_ref, t_scr):
    b = pl.program_id(0)

    @pl.when(b == 0)
    def _init_tri():
        mi = lax.broadcasted_iota(jnp.int32, (N, N), 0)
        ni = lax.broadcasted_iota(jnp.int32, (N, N), 1)
        t_scr[...] = (mi < ni).astype(jnp.float32)

    row = row_ref[0]
    maskr = (row > THR).astype(jnp.float32)
    rank = jnp.dot(maskr, t_scr[...], preferred_element_type=jnp.float32, precision=lax.Precision.HIGHEST)
    kio = lax.broadcasted_iota(jnp.int32, (K, 1), 0).astype(jnp.float32)
    nio = lax.broadcasted_iota(jnp.int32, (K, N), 1).astype(jnp.float32)
    oh = ((rank == kio) & (maskr > 0)).astype(jnp.float32)
    idx = jnp.sum(oh * nio, axis=1, keepdims=True)
    msk = jnp.sum(oh, axis=1, keepdims=True)
    idx_ref[0] = idx.astype(jnp.int32)
    mask_ref[0] = msk


def _neighbor_stage(adj, cur_idx):
    grid_spec = pltpu.PrefetchScalarGridSpec(
        num_scalar_prefetch=1,
        grid=(B,),
        in_specs=[
            pl.BlockSpec((1, 1, N), lambda b, cur: (b * N + cur[b], 0, 0)),
        ],
        out_specs=[
            pl.BlockSpec((1, K, 1), lambda b, cur: (b, 0, 0)),
            pl.BlockSpec((1, K, 1), lambda b, cur: (b, 0, 0)),
        ],
        scratch_shapes=[pltpu.VMEM((N, N), jnp.float32)],
    )
    idx, msk = pl.pallas_call(
        _stage1_body,
        grid_spec=grid_spec,
        out_shape=[
            jax.ShapeDtypeStruct((B, K, 1), jnp.int32),
            jax.ShapeDtypeStruct((B, K, 1), jnp.float32),
        ],
    )(cur_idx, adj.reshape(B * N, 1, N))
    return idx.reshape(B, K), msk.reshape(B, K)


def _stage2_body(cur_ref, spi_ref,
                 node_ref, ii_ref, ag_ref, wa_ref, va_ref,
                 adj_any, emb_any, label_ref, cs_ref, csf_ref,
                 sidx_r_ref, smask_r_ref, sidx_c_ref, smask_c_ref,
                 wg_ref, bg_ref, wv_ref,
                 state_ref, next_ref, prob_ref, nscore_ref, nll_ref,
                 adj_rows, emb_rows, sem_adj, sem_emb):
    b = pl.program_id(0)

    copies = []
    for k in range(K):
        si = spi_ref[b * K + k]
        ca = pltpu.make_async_copy(adj_any.at[b, si], adj_rows.at[k], sem_adj)
        ce = pltpu.make_async_copy(emb_any.at[b, si], emb_rows.at[k], sem_emb)
        ca.start()
        ce.start()
        copies.append((ca, ce))

    ns = node_ref[0]
    ii = ii_ref[0]
    ag = ag_ref[0]
    W = wa_ref[...]
    va = va_ref[...]

    def attn_e(x):
        t = jnp.tanh(jnp.dot(x, W, preferred_element_type=jnp.float32, precision=lax.Precision.HIGHEST))
        return jnp.sum(t * va, axis=1, keepdims=True)

    e0, e1, e2 = attn_e(ii), attn_e(ag), attn_e(ns)
    em = jnp.maximum(jnp.maximum(e0, e1), e2)
    x0, x1, x2 = jnp.exp(e0 - em), jnp.exp(e1 - em), jnp.exp(e2 - em)
    az = x0 + x1 + x2
    state = (x0 * ii + x1 * ag + x2 * ns) / az
    state_ref[0] = state

    sidx_c = sidx_c_ref[0]
    mask_c = smask_c_ref[0]
    sidx_r = sidx_r_ref[0]
    mask_r = smask_r_ref[0]
    nio_r = lax.broadcasted_iota(jnp.int32, (K, N), 1)
    oh_t = ((sidx_c == nio_r) & (mask_c > 0)).astype(jnp.float32)
    nio_c = lax.broadcasted_iota(jnp.int32, (N, K), 0)
    oh = ((nio_c == sidx_r) & (mask_r > 0)).astype(jnp.float32)

    label_col = label_ref[0]
    step_label = (jnp.dot(oh_t, label_col, preferred_element_type=jnp.float32, precision=lax.Precision.HIGHEST)
                  * (1.0 - SMOOTH) + SMOOTH / K)

    cs_col = cs_ref[0]
    total = jnp.sum(csf_ref[...], axis=1, keepdims=True)[:, :1]

    for ca, ce in copies:
        ce.wait()

    emb_g = emb_rows[...]
    se = jnp.concatenate(
        [emb_g, jnp.broadcast_to(state, (K, D))], axis=1)
    h = jnp.tanh(jnp.dot(se, wg_ref[...], preferred_element_type=jnp.float32, precision=lax.Precision.HIGHEST)
                 + bg_ref[...])

    for ca, ce in copies:
        ca.wait()

    adj_g = adj_rows[...]
    adj_sub = jnp.dot(adj_g, oh, preferred_element_type=jnp.float32, precision=lax.Precision.HIGHEST)
    step_adj = (adj_sub * (adj_sub > THR).astype(jnp.float32)
                * mask_c * mask_r)
    a_norm = step_adj / (jnp.sum(step_adj, axis=1, keepdims=True) + 1e-6)
    h2 = jnp.dot(a_norm, h, preferred_element_type=jnp.float32, precision=lax.Precision.HIGHEST) + h
    acts = jnp.sum(h2 * wv_ref[...], axis=1, keepdims=True)
    acts = jnp.where(mask_c > 0, acts, MASK_VAL)

    mx = jnp.max(acts, axis=0, keepdims=True)
    ex = jnp.exp(acts - mx)
    z = jnp.sum(ex, axis=0, keepdims=True)
    score = ex / z

    smax = jnp.max(score, axis=0, keepdims=True)
    kio = lax.broadcasted_iota(jnp.int32, (K, 1), 0)
    kstar = jnp.min(jnp.where(score == smax, kio, K), axis=0, keepdims=True)
    prob_ref[0] = jnp.log(smax + 1e-12)
    next_ref[0] = jnp.sum(jnp.where(kio == kstar, sidx_c, 0),
                          axis=0, keepdims=True)

    nz_cur = jnp.sum((cs_col != 0).astype(jnp.float32), axis=0, keepdims=True)
    nz_sc = jnp.sum((score != 0).astype(jnp.float32), axis=0, keepdims=True)
    tnz = total != 0
    cur2 = jnp.where(tnz, (1.0 - PROP) * cs_col * nz_cur, cs_col)
    sc2 = jnp.where(tnz, PROP * score * nz_sc, score)
    nsc = cur2 + jnp.dot(oh, sc2, preferred_element_type=jnp.float32, precision=lax.Precision.HIGHEST)
    nscore_ref[0] = nsc / jnp.sum(nsc, axis=0, keepdims=True)

    def log_t(u, t):
        return (jnp.power(u, 1.0 - t) - 1.0) / (1.0 - t)

    def exp_t(u, t):
        base = jnp.clip(1.0 + (1.0 - t) * u, 1e-6, None)
        return jnp.power(base, 1.0 / (1.0 - t))

    mu = jnp.max(acts, axis=0, keepdims=True)
    na = acts - mu
    for _ in range(5):
        zz = jnp.sum(exp_t(na, T2), axis=0, keepdims=True)
        na = (acts - mu) * jnp.power(zz, 1.0 - T2)
    zz = jnp.sum(exp_t(na, T2), axis=0, keepdims=True)
    norm_const = -log_t(1.0 / zz, T2) + mu
    probs = exp_t(acts - norm_const, T2)
    eps = 1e-10
    ll = (step_label * log_t(step_label + eps, T1)
          - step_label * log_t(probs + eps, T1)
          - jnp.power(step_label + eps, 2.0 - T1) / (2.0 - T1)
          + jnp.power(probs + eps, 2.0 - T1) / (2.0 - T1))
    part = jnp.sum(ll, axis=0, keepdims=True)

    @pl.when(b == 0)
    def _init_nll():
        nll_ref[...] = jnp.zeros_like(nll_ref)

    nll_ref[...] += part


def _main_stage(adj, label, emb, input_info, agent_state, cur_idx,
                current_score, sidx, smask, W_attn, v_attn, Wg, bg, wv):
    spi = sidx.reshape(B * K)
    grid_spec = pltpu.PrefetchScalarGridSpec(
        num_scalar_prefetch=2,
        grid=(B,),
        in_specs=[
            pl.BlockSpec((1, 1, D), lambda b, c, s: (b * N + c[b], 0, 0)),
            pl.BlockSpec((1, 1, D), lambda b, c, s: (b, 0, 0)),
            pl.BlockSpec((1, 1, D), lambda b, c, s: (b, 0, 0)),
            pl.BlockSpec((D, D), lambda b, c, s: (0, 0)),
            pl.BlockSpec((1, D), lambda b, c, s: (0, 0)),
            pl.BlockSpec(memory_space=pltpu.MemorySpace.HBM),
            pl.BlockSpec(memory_space=pltpu.MemorySpace.HBM),
            pl.BlockSpec((1, N, 1), lambda b, c, s: (b, 0, 0)),
            pl.BlockSpec((1, N, 1), lambda b, c, s: (b, 0, 0)),
            pl.BlockSpec((1, B * N), lambda b, c, s: (0, 0)),
            pl.BlockSpec((1, 1, K), lambda b, c, s: (b, 0, 0)),
            pl.BlockSpec((1, 1, K), lambda b, c, s: (b, 0, 0)),
            pl.BlockSpec((1, K, 1), lambda b, c, s: (b, 0, 0)),
            pl.BlockSpec((1, K, 1), lambda b, c, s: (b, 0, 0)),
            pl.BlockSpec((2 * D, D), lambda b, c, s: (0, 0)),
            pl.BlockSpec((1, D), lambda b, c, s: (0, 0)),
            pl.BlockSpec((1, D), lambda b, c, s: (0, 0)),
        ],
        out_specs=[
            pl.BlockSpec((1, 1, D), lambda b, c, s: (b, 0, 0)),
            pl.BlockSpec((1, 1, 1), lambda b, c, s: (b, 0, 0)),
            pl.BlockSpec((1, 1, 1), lambda b, c, s: (b, 0, 0)),
            pl.BlockSpec((1, N, 1), lambda b, c, s: (b, 0, 0)),
            pl.BlockSpec((1, 1), lambda b, c, s: (0, 0)),
        ],
        scratch_shapes=[
            pltpu.VMEM((K, N), jnp.float32),
            pltpu.VMEM((K, D), jnp.float32),
            pltpu.SemaphoreType.DMA,
            pltpu.SemaphoreType.DMA,
        ],
    )
    out = pl.pallas_call(
        _stage2_body,
        grid_spec=grid_spec,
        out_shape=[
            jax.ShapeDtypeStruct((B, 1, D), jnp.float32),
            jax.ShapeDtypeStruct((B, 1, 1), jnp.int32),
            jax.ShapeDtypeStruct((B, 1, 1), jnp.float32),
            jax.ShapeDtypeStruct((B, N, 1), jnp.float32),
            jax.ShapeDtypeStruct((1, 1), jnp.float32),
        ],
    )(cur_idx, spi,
      emb.reshape(B * N, 1, D),
      input_info.reshape(B, 1, D), agent_state.reshape(B, 1, D),
      W_attn, v_attn.reshape(1, D),
      adj, emb, label.reshape(B, N, 1), current_score.reshape(B, N, 1),
      current_score.reshape(1, B * N),
      sidx.reshape(B, 1, K), smask.reshape(B, 1, K),
      sidx.reshape(B, K, 1), smask.reshape(B, K, 1),
      Wg, bg.reshape(1, D), wv.reshape(1, D))
    state, next_node, prob, new_score, nll = out
    return (state.reshape(B, D), next_node.reshape(B), prob.reshape(B),
            new_score.reshape(B, N), nll.reshape(()))


@jax.jit
def kernel(adj, label, all_node_embedding, input_info, agent_state,
           current_node_idx, current_activation, current_score,
           W_attn, v_attn, Wg, bg, wv):
    cur_idx = current_node_idx.astype(jnp.int32)
    sidx, smask = _neighbor_stage(adj, cur_idx)
    state, next_node, prob, new_score, nll = _main_stage(
        adj, label, all_node_embedding, input_info, agent_state, cur_idx,
        current_score, sidx, smask, W_attn, v_attn, Wg, bg, wv)
    return (state, next_node, current_activation, new_score, prob, nll)
